# Initial kernel scaffold; baseline (speedup 1.0000x reference)
import jax
import jax.numpy as jnp
from jax.experimental import pallas as pl


def kernel(x, edge_index, W1, b1, W2, b2, W3, b3, Wc, bc):
    raise NotImplementedError("write your pallas kernel here")


# speedup vs baseline: 13.9216x; 13.9216x over previous
import functools

import jax
import jax.numpy as jnp
from jax import lax
from jax.experimental import pallas as pl
from jax.experimental.pallas import tpu as pltpu
from jax.experimental.pallas import tpu_sc as plsc

N = 10000
E = 320000
NC = 2
NS = 16
NW = NC * NS
CH = 128
NCH = 80
E_PAD = NW * NCH * CH
NROWS = 10240
RPW = NROWS // NS

_MESH = dict(core_axis_name="c", subcore_axis_name="s", num_cores=NC,
             num_subcores=NS)


def _scatter_rows(width):
    def body(table, srcs, dsts, zeros, out, src_v, dst_v, rows_v, acc):
        c = lax.axis_index("c")
        s = lax.axis_index("s")
        wid = c * NS + s
        pltpu.sync_copy(zeros.at[pl.ds(s * RPW, RPW)],
                        acc.at[pl.ds(s * RPW, RPW)])
        pltpu.sync_copy(srcs.at[wid], src_v)
        pltpu.sync_copy(dsts.at[wid], dst_v)
        plsc.subcore_barrier()

        def chunk(j, carry):
            pltpu.sync_copy(table.at[src_v.at[j]], rows_v)
            pltpu.sync_copy(rows_v, acc.at[dst_v.at[j]], add=True)
            return carry

        lax.fori_loop(0, NCH, chunk, 0)
        plsc.subcore_barrier()
        pltpu.sync_copy(acc.at[pl.ds(s * RPW, RPW)],
                        out.at[c, pl.ds(s * RPW, RPW)])

    return pl.kernel(
        body,
        out_type=jax.ShapeDtypeStruct((NC, NROWS, width), jnp.float32),
        mesh=plsc.VectorSubcoreMesh(**_MESH),
        compiler_params=pltpu.CompilerParams(use_tc_tiling_on_sc=False),
        scratch_types=[
            pltpu.VMEM((NCH, CH), jnp.int32),
            pltpu.VMEM((NCH, CH), jnp.int32),
            pltpu.VMEM((CH, width), jnp.float32),
            pltpu.VMEM_SHARED((NROWS, width), jnp.float32),
        ],
    )


def _degree_hist():
    width = 16

    def body(dsts, ones, zeros, out, dst_v, rows_v, acc):
        c = lax.axis_index("c")
        s = lax.axis_index("s")
        wid = c * NS + s
        pltpu.sync_copy(zeros.at[pl.ds(s * RPW, RPW)],
                        acc.at[pl.ds(s * RPW, RPW)])
        pltpu.sync_copy(dsts.at[wid], dst_v)
        pltpu.sync_copy(ones, rows_v)
        plsc.subcore_barrier()

        def chunk(j, carry):
            pltpu.sync_copy(rows_v, acc.at[dst_v.at[j]], add=True)
            return carry

        lax.fori_loop(0, NCH, chunk, 0)
        plsc.subcore_barrier()
        pltpu.sync_copy(acc.at[pl.ds(s * RPW, RPW)],
                        out.at[c, pl.ds(s * RPW, RPW)])

    return pl.kernel(
        body,
        out_type=jax.ShapeDtypeStruct((NC, NROWS, width), jnp.float32),
        mesh=plsc.VectorSubcoreMesh(**_MESH),
        compiler_params=pltpu.CompilerParams(use_tc_tiling_on_sc=False),
        scratch_types=[
            pltpu.VMEM((NCH, CH), jnp.int32),
            pltpu.VMEM((CH, width), jnp.float32),
            pltpu.VMEM_SHARED((NROWS, width), jnp.float32),
        ],
    )


def _tc_first(degp_ref, x_ref, w_ref, dis_ref, g_ref):
    deg = degp_ref[0][:N, :1] + degp_ref[1][:N, :1] + 1.0
    dis = lax.rsqrt(deg)
    dis_ref[...] = dis
    g_ref[...] = dis * jnp.dot(x_ref[...], w_ref[...],
                               preferred_element_type=jnp.float32)


def _tc_mid(sp_ref, g_ref, dis_ref, b_ref, w_ref, gout_ref):
    s = sp_ref[0][:N, :] + sp_ref[1][:N, :] + g_ref[...]
    dis = dis_ref[...]
    h = jnp.maximum(dis * s + b_ref[...], 0.0)
    gout_ref[...] = dis * jnp.dot(h, w_ref[...],
                                  preferred_element_type=jnp.float32)


def _tc_last(sp_ref, g_ref, dis_ref, b_ref, wc_ref, bc_ref, out_ref):
    s = sp_ref[0][:N, :] + sp_ref[1][:N, :] + g_ref[...]
    h = jnp.maximum(dis_ref[...] * s + b_ref[...], 0.0)
    out_ref[...] = jnp.dot(h, wc_ref[...],
                           preferred_element_type=jnp.float32) + bc_ref[...]


def _tc_call(body, out_shapes):
    return pl.pallas_call(body, out_shape=out_shapes)


def kernel(x, edge_index, W1, b1, W2, b2, W3, b3, Wc, bc):
    src = edge_index[0].astype(jnp.int32)
    dst = edge_index[1].astype(jnp.int32)
    pad = E_PAD - E
    src_p = jnp.concatenate([src, jnp.zeros((pad,), jnp.int32)])
    dst_p = jnp.concatenate([dst, jnp.full((pad,), N, jnp.int32)])
    srcs = src_p.reshape(NW, NCH, CH)
    dsts = dst_p.reshape(NW, NCH, CH)

    ones16 = jnp.ones((CH, 16), jnp.float32)
    z16 = jnp.zeros((NROWS, 16), jnp.float32)
    z64 = jnp.zeros((NROWS, 64), jnp.float32)
    z32 = jnp.zeros((NROWS, 32), jnp.float32)

    degp = _degree_hist()(dsts, ones16, z16)

    scat64 = _scatter_rows(64)
    scat32 = _scatter_rows(32)

    dis, g1 = _tc_call(_tc_first, (
        jax.ShapeDtypeStruct((N, 1), jnp.float32),
        jax.ShapeDtypeStruct((N, 64), jnp.float32),
    ))(degp, x, W1)

    s1 = scat64(g1, srcs, dsts, z64)
    g2 = _tc_call(_tc_mid, jax.ShapeDtypeStruct((N, 64), jnp.float32))(
        s1, g1, dis, b1.reshape(1, 64), W2)

    s2 = scat64(g2, srcs, dsts, z64)
    g3 = _tc_call(_tc_mid, jax.ShapeDtypeStruct((N, 32), jnp.float32))(
        s2, g2, dis, b2.reshape(1, 64), W3)

    s3 = scat32(g3, srcs, dsts, z32)
    out = _tc_call(_tc_last, jax.ShapeDtypeStruct((N, 1), jnp.float32))(
        s3, g3, dis, b3.reshape(1, 32), Wc, bc.reshape(1, 1))
    return out


# speedup vs baseline: 16.4914x; 1.1846x over previous
import functools

import jax
import jax.numpy as jnp
from jax import lax
from jax.experimental import pallas as pl
from jax.experimental.pallas import tpu as pltpu
from jax.experimental.pallas import tpu_sc as plsc

N = 10000
E = 320000
NC = 2
NS = 16
NW = NC * NS
CH = 128
NCH = 80
E_PAD = NW * NCH * CH
NROWS = 10240
RPW = NROWS // NS

_MESH = dict(core_axis_name="c", subcore_axis_name="s", num_cores=NC,
             num_subcores=NS)


def _scatter_rows(width):
    def body(table, srcs, dsts, zeros, out, src_v, dst_v, rows_v, acc,
             g0, g1, g2, g3, s0, s1, s2, s3):
        gs = [g0, g1, g2, g3]
        ss = [s0, s1, s2, s3]
        c = lax.axis_index("c")
        s = lax.axis_index("s")
        wid = c * NS + s
        pltpu.sync_copy(zeros.at[pl.ds(s * RPW, RPW)],
                        acc.at[pl.ds(s * RPW, RPW)])
        pltpu.sync_copy(srcs.at[wid], src_v)
        pltpu.sync_copy(dsts.at[wid], dst_v)
        plsc.subcore_barrier()

        def g_start(j, b):
            pltpu.async_copy(table.at[src_v.at[j]], rows_v.at[b], gs[b])

        def g_wait(j, b):
            pltpu.make_async_copy(table.at[src_v.at[j]], rows_v.at[b],
                                  gs[b]).wait()

        def s_start(j, b):
            pltpu.async_copy(rows_v.at[b], acc.at[dst_v.at[j]], ss[b],
                             add=True)

        def s_wait(j, b):
            pltpu.make_async_copy(rows_v.at[b], acc.at[dst_v.at[j]],
                                  ss[b]).wait()

        g_start(0, 0)
        g_start(1, 1)

        def outer(k, carry):
            for u in range(4):
                j = 4 * k + u
                b2 = (u + 2) % 4
                g_wait(j, u)
                s_start(j, u)

                @pl.when(j >= 2)
                def _():
                    s_wait(j - 2, b2)

                @pl.when(j + 2 < NCH)
                def _():
                    g_start(j + 2, b2)
            return carry

        lax.fori_loop(0, NCH // 4, outer, 0)
        s_wait(NCH - 2, (NCH - 2) % 4)
        s_wait(NCH - 1, (NCH - 1) % 4)
        plsc.subcore_barrier()
        pltpu.sync_copy(acc.at[pl.ds(s * RPW, RPW)],
                        out.at[c, pl.ds(s * RPW, RPW)])

    return pl.kernel(
        body,
        out_type=jax.ShapeDtypeStruct((NC, NROWS, width), jnp.float32),
        mesh=plsc.VectorSubcoreMesh(**_MESH),
        compiler_params=pltpu.CompilerParams(use_tc_tiling_on_sc=False),
        scratch_types=[
            pltpu.VMEM((NCH, CH), jnp.int32),
            pltpu.VMEM((NCH, CH), jnp.int32),
            pltpu.VMEM((4, CH, width), jnp.float32),
            pltpu.VMEM_SHARED((NROWS, width), jnp.float32),
            pltpu.SemaphoreType.DMA,
            pltpu.SemaphoreType.DMA,
            pltpu.SemaphoreType.DMA,
            pltpu.SemaphoreType.DMA,
            pltpu.SemaphoreType.DMA,
            pltpu.SemaphoreType.DMA,
            pltpu.SemaphoreType.DMA,
            pltpu.SemaphoreType.DMA,
        ],
    )


def _degree_hist():
    width = 16

    def body(dsts, ones, zeros, out, dst_v, rows_v, acc, s0, s1, s2, s3):
        ss = [s0, s1, s2, s3]
        c = lax.axis_index("c")
        s = lax.axis_index("s")
        wid = c * NS + s
        pltpu.sync_copy(zeros.at[pl.ds(s * RPW, RPW)],
                        acc.at[pl.ds(s * RPW, RPW)])
        pltpu.sync_copy(dsts.at[wid], dst_v)
        pltpu.sync_copy(ones, rows_v)
        plsc.subcore_barrier()

        def s_wait(j, b):
            pltpu.make_async_copy(rows_v, acc.at[dst_v.at[j]], ss[b]).wait()

        def outer(k, carry):
            for u in range(4):
                j = 4 * k + u

                @pl.when(j >= 4)
                def _():
                    s_wait(j - 4, u)

                pltpu.async_copy(rows_v, acc.at[dst_v.at[j]], ss[u],
                                 add=True)
            return carry

        lax.fori_loop(0, NCH // 4, outer, 0)
        for u in range(4):
            s_wait(NCH - 4 + u, u)
        plsc.subcore_barrier()
        pltpu.sync_copy(acc.at[pl.ds(s * RPW, RPW)],
                        out.at[c, pl.ds(s * RPW, RPW)])

    return pl.kernel(
        body,
        out_type=jax.ShapeDtypeStruct((NC, NROWS, width), jnp.float32),
        mesh=plsc.VectorSubcoreMesh(**_MESH),
        compiler_params=pltpu.CompilerParams(use_tc_tiling_on_sc=False),
        scratch_types=[
            pltpu.VMEM((NCH, CH), jnp.int32),
            pltpu.VMEM((CH, width), jnp.float32),
            pltpu.VMEM_SHARED((NROWS, width), jnp.float32),
            pltpu.SemaphoreType.DMA,
            pltpu.SemaphoreType.DMA,
            pltpu.SemaphoreType.DMA,
            pltpu.SemaphoreType.DMA,
        ],
    )


def _tc_first(degp_ref, x_ref, w_ref, dis_ref, g_ref):
    deg = degp_ref[0][:N, :1] + degp_ref[1][:N, :1] + 1.0
    dis = lax.rsqrt(deg)
    dis_ref[...] = dis
    g_ref[...] = dis * jnp.dot(x_ref[...], w_ref[...],
                               preferred_element_type=jnp.float32)


def _tc_mid(sp_ref, g_ref, dis_ref, b_ref, w_ref, gout_ref):
    s = sp_ref[0][:N, :] + sp_ref[1][:N, :] + g_ref[...]
    dis = dis_ref[...]
    h = jnp.maximum(dis * s + b_ref[...], 0.0)
    gout_ref[...] = dis * jnp.dot(h, w_ref[...],
                                  preferred_element_type=jnp.float32)


def _tc_last(sp_ref, g_ref, dis_ref, b_ref, wc_ref, bc_ref, out_ref):
    s = sp_ref[0][:N, :] + sp_ref[1][:N, :] + g_ref[...]
    h = jnp.maximum(dis_ref[...] * s + b_ref[...], 0.0)
    out_ref[...] = jnp.dot(h, wc_ref[...],
                           preferred_element_type=jnp.float32) + bc_ref[...]


def _tc_call(body, out_shapes):
    return pl.pallas_call(body, out_shape=out_shapes)


def kernel(x, edge_index, W1, b1, W2, b2, W3, b3, Wc, bc):
    src = edge_index[0].astype(jnp.int32)
    dst = edge_index[1].astype(jnp.int32)
    pad = E_PAD - E
    src_p = jnp.concatenate([src, jnp.zeros((pad,), jnp.int32)])
    dst_p = jnp.concatenate([dst, jnp.full((pad,), N, jnp.int32)])
    srcs = src_p.reshape(NW, NCH, CH)
    dsts = dst_p.reshape(NW, NCH, CH)

    ones16 = jnp.ones((CH, 16), jnp.float32)
    z16 = jnp.zeros((NROWS, 16), jnp.float32)
    z64 = jnp.zeros((NROWS, 64), jnp.float32)
    z32 = jnp.zeros((NROWS, 32), jnp.float32)

    degp = _degree_hist()(dsts, ones16, z16)

    scat64 = _scatter_rows(64)
    scat32 = _scatter_rows(32)

    dis, g1 = _tc_call(_tc_first, (
        jax.ShapeDtypeStruct((N, 1), jnp.float32),
        jax.ShapeDtypeStruct((N, 64), jnp.float32),
    ))(degp, x, W1)

    s1 = scat64(g1, srcs, dsts, z64)
    g2 = _tc_call(_tc_mid, jax.ShapeDtypeStruct((N, 64), jnp.float32))(
        s1, g1, dis, b1.reshape(1, 64), W2)

    s2 = scat64(g2, srcs, dsts, z64)
    g3 = _tc_call(_tc_mid, jax.ShapeDtypeStruct((N, 32), jnp.float32))(
        s2, g2, dis, b2.reshape(1, 64), W3)

    s3 = scat32(g3, srcs, dsts, z32)
    out = _tc_call(_tc_last, jax.ShapeDtypeStruct((N, 1), jnp.float32))(
        s3, g3, dis, b3.reshape(1, 32), Wc, bc.reshape(1, 1))
    return out


# speedup vs baseline: 16.6770x; 1.0113x over previous
import functools

import jax
import jax.numpy as jnp
from jax import lax
from jax.experimental import pallas as pl
from jax.experimental.pallas import tpu as pltpu
from jax.experimental.pallas import tpu_sc as plsc

N = 10000
E = 320000
NC = 2
NS = 16
NW = NC * NS
CH = 128
NCH = 80
E_PAD = NW * NCH * CH
NROWS = 10240
RPW = NROWS // NS

_MESH = dict(core_axis_name="c", subcore_axis_name="s", num_cores=NC,
             num_subcores=NS)


def _scatter_rows(width):
    def body(table, srcs, dsts, zeros, out, src_v, dst_v, rows_v, acc,
             g0, g1, g2, g3, s0, s1, s2, s3):
        gs = [g0, g1, g2, g3]
        ss = [s0, s1, s2, s3]
        c = lax.axis_index("c")
        s = lax.axis_index("s")
        wid = c * NS + s
        pltpu.sync_copy(zeros.at[pl.ds(s * RPW, RPW)],
                        acc.at[pl.ds(s * RPW, RPW)])
        pltpu.sync_copy(srcs.at[wid], src_v)
        pltpu.sync_copy(dsts.at[wid], dst_v)
        plsc.subcore_barrier()

        def g_start(j, b):
            pltpu.async_copy(table.at[src_v.at[j]], rows_v.at[b], gs[b])

        def g_wait(j, b):
            pltpu.make_async_copy(table.at[src_v.at[j]], rows_v.at[b],
                                  gs[b]).wait()

        def s_start(j, b):
            pltpu.async_copy(rows_v.at[b], acc.at[dst_v.at[j]], ss[b],
                             add=True)

        def s_wait(j, b):
            pltpu.make_async_copy(rows_v.at[b], acc.at[dst_v.at[j]],
                                  ss[b]).wait()

        g_start(0, 0)
        g_start(1, 1)

        def outer(k, carry):
            for u in range(4):
                j = 4 * k + u
                b2 = (u + 2) % 4
                g_wait(j, u)
                s_start(j, u)

                @pl.when(j >= 2)
                def _():
                    s_wait(j - 2, b2)

                @pl.when(j + 2 < NCH)
                def _():
                    g_start(j + 2, b2)
            return carry

        lax.fori_loop(0, NCH // 4, outer, 0)
        s_wait(NCH - 2, (NCH - 2) % 4)
        s_wait(NCH - 1, (NCH - 1) % 4)
        plsc.subcore_barrier()
        pltpu.sync_copy(acc.at[pl.ds(s * RPW, RPW)],
                        out.at[c, pl.ds(s * RPW, RPW)])

    return pl.kernel(
        body,
        out_type=jax.ShapeDtypeStruct((NC, NROWS, width), jnp.float32),
        mesh=plsc.VectorSubcoreMesh(**_MESH),
        compiler_params=pltpu.CompilerParams(use_tc_tiling_on_sc=False),
        scratch_types=[
            pltpu.VMEM((NCH, CH), jnp.int32),
            pltpu.VMEM((NCH, CH), jnp.int32),
            pltpu.VMEM((4, CH, width), jnp.float32),
            pltpu.VMEM_SHARED((NROWS, width), jnp.float32),
            pltpu.SemaphoreType.DMA,
            pltpu.SemaphoreType.DMA,
            pltpu.SemaphoreType.DMA,
            pltpu.SemaphoreType.DMA,
            pltpu.SemaphoreType.DMA,
            pltpu.SemaphoreType.DMA,
            pltpu.SemaphoreType.DMA,
            pltpu.SemaphoreType.DMA,
        ],
    )


def _degree_hist():
    width = 16

    def body(dsts, ones, zeros, out, dst_v, rows_v, acc, s0, s1, s2, s3):
        ss = [s0, s1, s2, s3]
        c = lax.axis_index("c")
        s = lax.axis_index("s")
        wid = c * NS + s
        pltpu.sync_copy(zeros.at[pl.ds(s * RPW, RPW)],
                        acc.at[pl.ds(s * RPW, RPW)])
        pltpu.sync_copy(dsts.at[wid], dst_v)
        pltpu.sync_copy(ones, rows_v)
        plsc.subcore_barrier()

        def s_wait(j, b):
            pltpu.make_async_copy(rows_v, acc.at[dst_v.at[j]], ss[b]).wait()

        def outer(k, carry):
            for u in range(4):
                j = 4 * k + u

                @pl.when(j >= 4)
                def _():
                    s_wait(j - 4, u)

                pltpu.async_copy(rows_v, acc.at[dst_v.at[j]], ss[u],
                                 add=True)
            return carry

        lax.fori_loop(0, NCH // 4, outer, 0)
        for u in range(4):
            s_wait(NCH - 4 + u, u)
        plsc.subcore_barrier()
        pltpu.sync_copy(acc.at[pl.ds(s * RPW, RPW)],
                        out.at[c, pl.ds(s * RPW, RPW)])

    return pl.kernel(
        body,
        out_type=jax.ShapeDtypeStruct((NC, NROWS, width), jnp.float32),
        mesh=plsc.VectorSubcoreMesh(**_MESH),
        compiler_params=pltpu.CompilerParams(use_tc_tiling_on_sc=False),
        scratch_types=[
            pltpu.VMEM((NCH, CH), jnp.int32),
            pltpu.VMEM((CH, width), jnp.float32),
            pltpu.VMEM_SHARED((NROWS, width), jnp.float32),
            pltpu.SemaphoreType.DMA,
            pltpu.SemaphoreType.DMA,
            pltpu.SemaphoreType.DMA,
            pltpu.SemaphoreType.DMA,
        ],
    )


def _tc_first(degp_ref, x_ref, w_ref, dis_ref, g_ref):
    deg = degp_ref[0][:N, :1] + degp_ref[1][:N, :1] + 1.0
    dis = lax.rsqrt(deg)
    dis_ref[...] = dis
    g_ref[...] = dis * jnp.dot(x_ref[...], w_ref[...],
                               preferred_element_type=jnp.float32)


def _tc_mid(sp_ref, g_ref, dis_ref, b_ref, w_ref, gout_ref):
    s = sp_ref[0][:N, :] + sp_ref[1][:N, :] + g_ref[...]
    dis = dis_ref[...]
    h = jnp.maximum(dis * s + b_ref[...], 0.0)
    gout_ref[...] = dis * jnp.dot(h, w_ref[...],
                                  preferred_element_type=jnp.float32)


def _tc_last(sp_ref, g_ref, dis_ref, b_ref, wc_ref, bc_ref, out_ref):
    s = sp_ref[0][:N, :] + sp_ref[1][:N, :] + g_ref[...]
    h = jnp.maximum(dis_ref[...] * s + b_ref[...], 0.0)
    out_ref[...] = jnp.dot(h, wc_ref[...],
                           preferred_element_type=jnp.float32) + bc_ref[...]


def _tc_call(body, out_shapes):
    return pl.pallas_call(body, out_shape=out_shapes)


def kernel(x, edge_index, W1, b1, W2, b2, W3, b3, Wc, bc):
    src = edge_index[0].astype(jnp.int32)
    dst = edge_index[1].astype(jnp.int32)
    pad = E_PAD - E
    src_p = jnp.concatenate([src, jnp.zeros((pad,), jnp.int32)])
    trash = N + jnp.arange(pad, dtype=jnp.int32) % (NROWS - N)
    dst_p = jnp.concatenate([dst, trash])
    srcs = src_p.reshape(NW, NCH, CH)
    dsts = dst_p.reshape(NW, NCH, CH)

    ones16 = jnp.ones((CH, 16), jnp.float32)
    z16 = jnp.zeros((NROWS, 16), jnp.float32)
    z64 = jnp.zeros((NROWS, 64), jnp.float32)
    z32 = jnp.zeros((NROWS, 32), jnp.float32)

    degp = _degree_hist()(dsts, ones16, z16)

    scat64 = _scatter_rows(64)
    scat32 = _scatter_rows(32)

    dis, g1 = _tc_call(_tc_first, (
        jax.ShapeDtypeStruct((N, 1), jnp.float32),
        jax.ShapeDtypeStruct((N, 64), jnp.float32),
    ))(degp, x, W1)

    s1 = scat64(g1, srcs, dsts, z64)
    g2 = _tc_call(_tc_mid, jax.ShapeDtypeStruct((N, 64), jnp.float32))(
        s1, g1, dis, b1.reshape(1, 64), W2)

    s2 = scat64(g2, srcs, dsts, z64)
    g3 = _tc_call(_tc_mid, jax.ShapeDtypeStruct((N, 32), jnp.float32))(
        s2, g2, dis, b2.reshape(1, 64), W3)

    s3 = scat32(g3, srcs, dsts, z32)
    out = _tc_call(_tc_last, jax.ShapeDtypeStruct((N, 1), jnp.float32))(
        s3, g3, dis, b3.reshape(1, 32), Wc, bc.reshape(1, 1))
    return out
